# baseline (speedup 1.0000x reference)
import functools

import jax
import jax.numpy as jnp
from jax import lax
from jax.experimental import pallas as pl
from jax.experimental.pallas import tpu as pltpu
from jax.experimental.pallas import tpu_sc as plsc

_N_WORKERS = 32
_BLK = 128


@functools.lru_cache(maxsize=None)
def _make(n_s: int, n_b: int, dim: int):
    total = n_s * n_b
    n_blocks = total // _BLK
    per_w = n_blocks // _N_WORKERS
    bt_per_s = n_b // _BLK
    dg = dim // 8
    assert per_w * _N_WORKERS == n_blocks
    mesh = plsc.VectorSubcoreMesh(core_axis_name="c", subcore_axis_name="s")

    @functools.partial(
        pl.kernel,
        out_type=jax.ShapeDtypeStruct((n_s, dg, bt_per_s, 8, _BLK),
                                      jnp.float32),
        mesh=mesh,
        scratch_types=[
            pltpu.VMEM((per_w * _BLK,), jnp.int32),
            [pltpu.VMEM((_BLK, dim), jnp.float32) for _ in range(2)],
            pltpu.VMEM((dim, _BLK), jnp.float32),
            [pltpu.SemaphoreType.DMA for _ in range(2)],
            pltpu.SemaphoreType.DMA,
        ],
        compiler_params=pltpu.CompilerParams(use_tc_tiling_on_sc=False, needs_layout_passes=False),
    )
    def gather_kernel(idx_hbm, table_hbm, out_hbm, idx_all, rows, trans_v,
                      gsem, wsem):
        wid = lax.axis_index("s") * 2 + lax.axis_index("c")
        wblk = wid * per_w

        pltpu.sync_copy(idx_hbm.at[pl.ds(wblk * _BLK, per_w * _BLK)], idx_all)

        def start_gather(par, j):
            pltpu.async_copy(
                table_hbm.at[idx_all.at[pl.ds(j * _BLK, _BLK)]],
                rows[par],
                gsem[par],
            )

        def wait_gather(par):
            pltpu.make_async_copy(
                table_hbm.at[idx_all.at[pl.ds(0, _BLK)]],
                rows[par], gsem[par]).wait()

        lane = lax.iota(jnp.int32, 16)
        diag = [(lane + j) % 16 for j in range(16)]

        def transpose_block(par):
            def bt_body(b0, carry):
                row_idx = lane + b0
                cols = [diag[j] + dt * 16
                        for dt in range(dim // 16) for j in range(16)]
                vals = [plsc.load_gather(rows[par], [row_idx, c])
                        for c in cols]
                for c, v in zip(cols, vals):
                    plsc.store_scatter(trans_v, [c, row_idx], v)
                return carry
            lax.fori_loop(0, _BLK // 16, lambda i, c: bt_body(i * 16, c), 0)

        def write_block(j):
            blk = wblk + j
            s = blk // bt_per_s
            bt = blk % bt_per_s
            copies = [
                pltpu.async_copy(trans_v.at[pl.ds(g8 * 8, 8), pl.ds(0, _BLK)],
                                 out_hbm.at[s, g8, bt], wsem)
                for g8 in range(dg)
            ]
            for c in copies:
                c.wait()

        start_gather(0, 0)
        start_gather(1, 1)

        def body(i, carry):
            for par in range(2):
                j = i * 2 + par
                wait_gather(par)
                transpose_block(par)
                start_gather(par, j + 2)
                write_block(j)
            return carry

        lax.fori_loop(0, per_w // 2 - 1, body, 0)

        for par in range(2):
            j = per_w - 2 + par
            wait_gather(par)
            transpose_block(par)
            write_block(j)

    return gather_kernel


def kernel(z, table):
    b, s = z.shape
    dim = table.shape[1]
    zf = z.T.reshape(b * s).astype(jnp.int32)
    out5 = _make(s, b, dim)(zf, table)
    return out5.transpose(2, 4, 0, 1, 3).reshape(b, s, dim)


# speedup vs baseline: 1.0269x; 1.0269x over previous
import functools

import jax
import jax.numpy as jnp
from jax import lax
from jax.experimental import pallas as pl
from jax.experimental.pallas import tpu as pltpu
from jax.experimental.pallas import tpu_sc as plsc

_N_WORKERS = 32
_BLK = 128


@functools.lru_cache(maxsize=None)
def _make(n_s: int, n_b: int, dim: int):
    total = n_s * n_b
    n_blocks = total // _BLK
    per_w = n_blocks // _N_WORKERS
    bt_per_s = n_b // _BLK
    dg = dim // 8
    assert per_w * _N_WORKERS == n_blocks
    mesh = plsc.VectorSubcoreMesh(core_axis_name="c", subcore_axis_name="s")

    @functools.partial(
        pl.kernel,
        out_type=jax.ShapeDtypeStruct((n_s, dg, bt_per_s, 8, _BLK),
                                      jnp.float32),
        mesh=mesh,
        scratch_types=[
            pltpu.VMEM((per_w * _BLK,), jnp.int32),
            [pltpu.VMEM((_BLK, dim), jnp.float32) for _ in range(2)],
            pltpu.VMEM((dim, _BLK), jnp.float32),
            [pltpu.SemaphoreType.DMA for _ in range(2)],
            pltpu.SemaphoreType.DMA,
        ],
        compiler_params=pltpu.CompilerParams(use_tc_tiling_on_sc=False, needs_layout_passes=False),
    )
    def gather_kernel(idx_hbm, table_hbm, out_hbm, idx_all, rows, trans_v,
                      gsem, wsem):
        wid = lax.axis_index("s") * 2 + lax.axis_index("c")
        wblk = wid * per_w

        pltpu.sync_copy(idx_hbm.at[pl.ds(wblk * _BLK, per_w * _BLK)], idx_all)

        def start_gather(par, j):
            pltpu.async_copy(
                table_hbm.at[idx_all.at[pl.ds(j * _BLK, _BLK)]],
                rows[par],
                gsem[par],
            )

        def wait_gather(par):
            pltpu.make_async_copy(
                table_hbm.at[idx_all.at[pl.ds(0, _BLK)]],
                rows[par], gsem[par]).wait()

        lane = lax.iota(jnp.int32, 16)
        diag = [(lane + j) % 16 for j in range(16)]

        def transpose_block(par):
            def bt_body(b0, carry):
                row_idx = lane + b0
                for dt in range(dim // 16):
                    cols = [diag[j] + dt * 16 for j in range(16)]
                    vals = [plsc.load_gather(rows[par], [row_idx, c])
                            for c in cols]
                    for c, v in zip(cols, vals):
                        plsc.store_scatter(trans_v, [c, row_idx], v)
                return carry
            lax.fori_loop(0, _BLK // 16, lambda i, c: bt_body(i * 16, c), 0)

        def write_block(j):
            blk = wblk + j
            s = blk // bt_per_s
            bt = blk % bt_per_s
            copies = [
                pltpu.async_copy(trans_v.at[pl.ds(g8 * 8, 8), pl.ds(0, _BLK)],
                                 out_hbm.at[s, g8, bt], wsem)
                for g8 in range(dg)
            ]
            for c in copies:
                c.wait()

        start_gather(0, 0)
        start_gather(1, 1)

        def body(i, carry):
            for par in range(2):
                j = i * 2 + par
                wait_gather(par)
                transpose_block(par)
                start_gather(par, j + 2)
                write_block(j)
            return carry

        lax.fori_loop(0, per_w // 2 - 1, body, 0)

        for par in range(2):
            j = per_w - 2 + par
            wait_gather(par)
            transpose_block(par)
            write_block(j)

    return gather_kernel


def kernel(z, table):
    b, s = z.shape
    dim = table.shape[1]
    zf = z.T.reshape(b * s).astype(jnp.int32)
    out5 = _make(s, b, dim)(zf, table)
    return out5.transpose(2, 4, 0, 1, 3).reshape(b, s, dim)
